# Initial kernel scaffold; baseline (speedup 1.0000x reference)
import functools

import jax
import jax.numpy as jnp
from jax import lax
from jax.experimental import pallas as pl
from jax.experimental.pallas import tpu as pltpu
from jax.experimental.pallas import tpu_sc as plsc

_N = 100000
_E = 3200000
_F = 128
_H = 16
_C = 18

_NC = 2
_NS = 16
_NW = _NC * _NS

_BE = 2000
_EPW = _E // _NW
_ECH = _EPW // _BE
_BN = 2000
_NCH = _N // _BN
_NCH_PT = -(-_NCH // _NS)

_BLK = 5000
_G = _N // _BLK


def _mesh():
    return plsc.VectorSubcoreMesh(core_axis_name="c", subcore_axis_name="s")


@functools.partial(
    pl.kernel,
    out_type=jax.ShapeDtypeStruct((_NC, _N), jnp.float32),
    mesh=_mesh(),
    scratch_types=[
        pltpu.VMEM((_BE,), jnp.int32),
        pltpu.VMEM((_BN,), jnp.float32),
        pltpu.VMEM_SHARED((_N,), jnp.float32),
        pltpu.SemaphoreType.DMA,
    ],
)
def _sc_deg(dst_hbm, out_hbm, idx_v, val_v, acc_sp, sem):
    c = lax.axis_index("c")
    s = lax.axis_index("s")
    wid = s * _NC + c

    def fill(i, carry):
        val_v[pl.ds(i * 16, 16)] = jnp.full((16,), carry, jnp.float32)
        return carry

    lax.fori_loop(0, _BN // 16, fill, jnp.float32(0.0))

    def zero_chunk(j, carry):
        ch = s + j * _NS

        @pl.when(ch < _NCH)
        def _():
            pltpu.sync_copy(val_v, acc_sp.at[pl.ds(ch * _BN, _BN)])

        return carry

    lax.fori_loop(0, _NCH_PT, zero_chunk, 0)
    plsc.subcore_barrier()

    lax.fori_loop(0, _BN // 16, fill, jnp.float32(1.0))

    def body(j, carry):
        off = wid * _EPW + j * _BE
        pltpu.sync_copy(dst_hbm.at[pl.ds(off, _BE)], idx_v)
        pltpu.sync_copy(val_v.at[pl.ds(0, _BE)], acc_sp.at[idx_v], add=True)
        return carry

    lax.fori_loop(0, _ECH, body, 0)
    plsc.subcore_barrier()

    def writeback(j, carry):
        ch = s + j * _NS

        @pl.when(ch < _NCH)
        def _():
            pltpu.sync_copy(acc_sp.at[pl.ds(ch * _BN, _BN)], val_v)
            pltpu.sync_copy(val_v, out_hbm.at[c, pl.ds(ch * _BN, _BN)])

        return carry

    lax.fori_loop(0, _NCH_PT, writeback, 0)


@functools.partial(
    pl.kernel,
    out_type=jax.ShapeDtypeStruct((_NC, _N, _H), jnp.float32),
    mesh=_mesh(),
    scratch_types=[
        pltpu.VMEM((_BE,), jnp.int32),
        pltpu.VMEM((_BE,), jnp.int32),
        pltpu.VMEM((_BE, _H), jnp.float32),
        pltpu.VMEM_SHARED((_N, _H), jnp.float32),
        pltpu.SemaphoreType.DMA,
    ],
)
def _sc_agg(tbl_hbm, src_hbm, dst_hbm, out_hbm, src_v, dst_v, rows_v, acc_sp,
            sem):
    c = lax.axis_index("c")
    s = lax.axis_index("s")
    wid = s * _NC + c

    def zrow(i, carry):
        rows_v[i, :] = jnp.zeros((16,), jnp.float32)
        return carry

    lax.fori_loop(0, _BE, zrow, 0)

    def zero_chunk(j, carry):
        ch = s + j * _NS

        @pl.when(ch < _NCH)
        def _():
            pltpu.sync_copy(rows_v.at[pl.ds(0, _BN)],
                            acc_sp.at[pl.ds(ch * _BN, _BN)])

        return carry

    lax.fori_loop(0, _NCH_PT, zero_chunk, 0)
    plsc.subcore_barrier()

    def body(j, carry):
        off = wid * _EPW + j * _BE
        pltpu.sync_copy(src_hbm.at[pl.ds(off, _BE)], src_v)
        pltpu.sync_copy(dst_hbm.at[pl.ds(off, _BE)], dst_v)
        pltpu.async_copy(tbl_hbm.at[src_v], rows_v, sem).wait()
        pltpu.sync_copy(rows_v, acc_sp.at[dst_v], add=True)
        return carry

    lax.fori_loop(0, _ECH, body, 0)
    plsc.subcore_barrier()

    def writeback(j, carry):
        ch = s + j * _NS

        @pl.when(ch < _NCH)
        def _():
            pltpu.sync_copy(acc_sp.at[pl.ds(ch * _BN, _BN)],
                            rows_v.at[pl.ds(0, _BN)])
            pltpu.sync_copy(rows_v.at[pl.ds(0, _BN)],
                            out_hbm.at[c, pl.ds(ch * _BN, _BN)])

        return carry

    lax.fori_loop(0, _NCH_PT, writeback, 0)


def _k1_body(x_ref, w1_ref, b1_ref, wc_ref, deg_ref, a1p_ref, dinv_ref):
    h0 = jnp.maximum(
        jnp.dot(x_ref[...], w1_ref[...], preferred_element_type=jnp.float32)
        + b1_ref[...], 0.0)
    deg = deg_ref[0] + deg_ref[1] + 1.0
    dinv = lax.rsqrt(deg)
    a1 = jnp.dot(h0, wc_ref[...], preferred_element_type=jnp.float32)
    a1p_ref[...] = dinv * a1
    dinv_ref[...] = dinv


def _tc_k1(x, W1, b1, Wc0, deg_p):
    return pl.pallas_call(
        _k1_body,
        grid=(_G,),
        in_specs=[
            pl.BlockSpec((_BLK, _F), lambda i: (i, 0)),
            pl.BlockSpec((_F, _H), lambda i: (0, 0)),
            pl.BlockSpec((1, _H), lambda i: (0, 0)),
            pl.BlockSpec((_H, _H), lambda i: (0, 0)),
            pl.BlockSpec((_NC, _BLK, 1), lambda i: (0, i, 0)),
        ],
        out_specs=[
            pl.BlockSpec((_BLK, _H), lambda i: (i, 0)),
            pl.BlockSpec((_BLK, 1), lambda i: (i, 0)),
        ],
        out_shape=[
            jax.ShapeDtypeStruct((_N, _H), jnp.float32),
            jax.ShapeDtypeStruct((_N, 1), jnp.float32),
        ],
    )(x, W1, b1, Wc0, deg_p)


def _k2_body(sp_ref, a1p_ref, dinv_ref, wc_ref, bc_ref, a2p_ref):
    dinv = dinv_ref[...]
    h1 = jnp.maximum(
        dinv * (sp_ref[0] + sp_ref[1] + a1p_ref[...]) + bc_ref[...], 0.0)
    a2 = jnp.dot(h1, wc_ref[...], preferred_element_type=jnp.float32)
    a2p_ref[...] = dinv * a2


def _tc_k2(s_p, a1p, dinv, Wc1, bc0):
    return pl.pallas_call(
        _k2_body,
        grid=(_G,),
        in_specs=[
            pl.BlockSpec((_NC, _BLK, _H), lambda i: (0, i, 0)),
            pl.BlockSpec((_BLK, _H), lambda i: (i, 0)),
            pl.BlockSpec((_BLK, 1), lambda i: (i, 0)),
            pl.BlockSpec((_H, _H), lambda i: (0, 0)),
            pl.BlockSpec((1, _H), lambda i: (0, 0)),
        ],
        out_specs=pl.BlockSpec((_BLK, _H), lambda i: (i, 0)),
        out_shape=jax.ShapeDtypeStruct((_N, _H), jnp.float32),
    )(s_p, a1p, dinv, Wc1, bc0)


def _k3_body(sp_ref, a2p_ref, dinv_ref, bc_ref, w2_ref, b2_ref, out_ref):
    dinv = dinv_ref[...]
    h2 = jnp.maximum(
        dinv * (sp_ref[0] + sp_ref[1] + a2p_ref[...]) + bc_ref[...], 0.0)
    logits = jnp.dot(h2, w2_ref[...],
                     preferred_element_type=jnp.float32) + b2_ref[...]
    m = jnp.max(logits, axis=-1, keepdims=True)
    e = jnp.exp(logits - m)
    lse = m + jnp.log(jnp.sum(e, axis=-1, keepdims=True))
    out_ref[...] = logits - lse


def _tc_k3(s_p, a2p, dinv, bc1, W2, b2):
    return pl.pallas_call(
        _k3_body,
        grid=(_G,),
        in_specs=[
            pl.BlockSpec((_NC, _BLK, _H), lambda i: (0, i, 0)),
            pl.BlockSpec((_BLK, _H), lambda i: (i, 0)),
            pl.BlockSpec((_BLK, 1), lambda i: (i, 0)),
            pl.BlockSpec((1, _H), lambda i: (0, 0)),
            pl.BlockSpec((_H, _C), lambda i: (0, 0)),
            pl.BlockSpec((1, _C), lambda i: (0, 0)),
        ],
        out_specs=pl.BlockSpec((_BLK, _C), lambda i: (i, 0)),
        out_shape=jax.ShapeDtypeStruct((_N, _C), jnp.float32),
    )(s_p, a2p, dinv, bc1, W2, b2)


def kernel(x, edge_index, edge_weight, W1, b1, Wc0, bc0, Wc1, bc1, W2, b2):
    src = edge_index[0]
    dst = edge_index[1]
    deg_p = _sc_deg(dst).reshape(_NC, _N, 1)
    a1p, dinv = _tc_k1(x, W1, b1.reshape(1, _H), Wc0, deg_p)
    s1p = _sc_agg(a1p, src, dst)
    a2p = _tc_k2(s1p, a1p, dinv, Wc1, bc0.reshape(1, _H))
    s2p = _sc_agg(a2p, src, dst)
    return _tc_k3(s2p, a2p, dinv, bc1.reshape(1, _H), W2, b2.reshape(1, _C))


# speedup vs baseline: 57.8998x; 57.8998x over previous
import functools

import jax
import jax.numpy as jnp
from jax import lax
from jax.experimental import pallas as pl
from jax.experimental.pallas import tpu as pltpu
from jax.experimental.pallas import tpu_sc as plsc

_N = 100000
_E = 3200000
_F = 128
_H = 16
_C = 18

_NC = 2
_NS = 16
_NW = _NC * _NS

_BE = 1000
_EPW = _E // _NW
_ECH = _EPW // _BE
_BN = 1000
_NCH = _N // _BN
_NCH_PT = -(-_NCH // _NS)

_BLK = 5000
_G = _N // _BLK


def _mesh():
    return plsc.VectorSubcoreMesh(core_axis_name="c", subcore_axis_name="s")


@functools.partial(
    pl.kernel,
    out_type=jax.ShapeDtypeStruct((_NC * _N,), jnp.float32),
    mesh=_mesh(),
    scratch_types=[
        pltpu.VMEM((_BE,), jnp.int32),
        pltpu.VMEM((_BN,), jnp.float32),
        pltpu.VMEM_SHARED((_N,), jnp.float32),
        pltpu.SemaphoreType.DMA,
    ],
)
def _sc_deg(dst_hbm, out_hbm, idx_v, val_v, acc_sp, sem):
    c = lax.axis_index("c")
    s = lax.axis_index("s")
    wid = s * _NC + c

    def fill(i, carry):
        val_v[pl.ds(i * 16, 16)] = jnp.full((16,), carry, jnp.float32)
        return carry

    lax.fori_loop(0, _BN // 16, fill, jnp.float32(0.0))

    def zero_chunk(j, carry):
        ch = s + j * _NS

        @pl.when(ch < _NCH)
        def _():
            pltpu.sync_copy(val_v, acc_sp.at[pl.ds(ch * _BN, _BN)])

        return carry

    lax.fori_loop(0, _NCH_PT, zero_chunk, 0)
    plsc.subcore_barrier()

    lax.fori_loop(0, _BN // 16, fill, jnp.float32(1.0))

    def body(j, carry):
        off = wid * _EPW + j * _BE
        pltpu.sync_copy(dst_hbm.at[pl.ds(off, _BE)], idx_v)
        pltpu.sync_copy(val_v.at[pl.ds(0, _BE)], acc_sp.at[idx_v], add=True)
        return carry

    lax.fori_loop(0, _ECH, body, 0)
    plsc.subcore_barrier()

    def writeback(j, carry):
        ch = s + j * _NS

        @pl.when(ch < _NCH)
        def _():
            pltpu.sync_copy(acc_sp.at[pl.ds(ch * _BN, _BN)], val_v)
            pltpu.sync_copy(val_v, out_hbm.at[pl.ds(c * _N + ch * _BN, _BN)])

        return carry

    lax.fori_loop(0, _NCH_PT, writeback, 0)


@functools.partial(
    pl.kernel,
    out_type=jax.ShapeDtypeStruct((_NC, _N, _H), jnp.float32),
    mesh=_mesh(),
    compiler_params=pltpu.CompilerParams(use_tc_tiling_on_sc=False),
    scratch_types=[
        pltpu.VMEM((_BE,), jnp.int32),
        pltpu.VMEM((_BE,), jnp.int32),
        pltpu.VMEM((_BE, _H), jnp.float32),
        pltpu.VMEM_SHARED((_N, _H), jnp.float32),
        pltpu.SemaphoreType.DMA,
    ],
)
def _sc_agg(tbl_hbm, src_hbm, dst_hbm, out_hbm, src_v, dst_v, rows_v, acc_sp,
            sem):
    c = lax.axis_index("c")
    s = lax.axis_index("s")
    wid = s * _NC + c

    def zrow(i, carry):
        rows_v[i, :] = jnp.zeros((16,), jnp.float32)
        return carry

    lax.fori_loop(0, _BE, zrow, 0)

    def zero_chunk(j, carry):
        ch = s + j * _NS

        @pl.when(ch < _NCH)
        def _():
            pltpu.sync_copy(rows_v.at[pl.ds(0, _BN)],
                            acc_sp.at[pl.ds(ch * _BN, _BN)])

        return carry

    lax.fori_loop(0, _NCH_PT, zero_chunk, 0)
    plsc.subcore_barrier()

    def body(j, carry):
        off = wid * _EPW + j * _BE
        pltpu.sync_copy(src_hbm.at[pl.ds(off, _BE)], src_v)
        pltpu.sync_copy(dst_hbm.at[pl.ds(off, _BE)], dst_v)
        pltpu.async_copy(tbl_hbm.at[src_v], rows_v, sem).wait()
        pltpu.sync_copy(rows_v, acc_sp.at[dst_v], add=True)
        return carry

    lax.fori_loop(0, _ECH, body, 0)
    plsc.subcore_barrier()

    def writeback(j, carry):
        ch = s + j * _NS

        @pl.when(ch < _NCH)
        def _():
            pltpu.sync_copy(acc_sp.at[pl.ds(ch * _BN, _BN)],
                            rows_v.at[pl.ds(0, _BN)])
            pltpu.sync_copy(rows_v.at[pl.ds(0, _BN)],
                            out_hbm.at[c, pl.ds(ch * _BN, _BN)])

        return carry

    lax.fori_loop(0, _NCH_PT, writeback, 0)


def _k1_body(x_ref, w1_ref, b1_ref, wc_ref, deg_ref, a1p_ref, dinv_ref):
    h0 = jnp.maximum(
        jnp.dot(x_ref[...], w1_ref[...], preferred_element_type=jnp.float32)
        + b1_ref[...], 0.0)
    deg = deg_ref[0] + deg_ref[1] + 1.0
    dinv = lax.rsqrt(deg)
    a1 = jnp.dot(h0, wc_ref[...], preferred_element_type=jnp.float32)
    a1p_ref[...] = dinv * a1
    dinv_ref[...] = dinv


def _tc_k1(x, W1, b1, Wc0, deg_p):
    return pl.pallas_call(
        _k1_body,
        grid=(_G,),
        in_specs=[
            pl.BlockSpec((_BLK, _F), lambda i: (i, 0)),
            pl.BlockSpec((_F, _H), lambda i: (0, 0)),
            pl.BlockSpec((1, _H), lambda i: (0, 0)),
            pl.BlockSpec((_H, _H), lambda i: (0, 0)),
            pl.BlockSpec((_NC, _BLK, 1), lambda i: (0, i, 0)),
        ],
        out_specs=[
            pl.BlockSpec((_BLK, _H), lambda i: (i, 0)),
            pl.BlockSpec((_BLK, 1), lambda i: (i, 0)),
        ],
        out_shape=[
            jax.ShapeDtypeStruct((_N, _H), jnp.float32),
            jax.ShapeDtypeStruct((_N, 1), jnp.float32),
        ],
    )(x, W1, b1, Wc0, deg_p)


def _k2_body(sp_ref, a1p_ref, dinv_ref, wc_ref, bc_ref, a2p_ref):
    dinv = dinv_ref[...]
    h1 = jnp.maximum(
        dinv * (sp_ref[0] + sp_ref[1] + a1p_ref[...]) + bc_ref[...], 0.0)
    a2 = jnp.dot(h1, wc_ref[...], preferred_element_type=jnp.float32)
    a2p_ref[...] = dinv * a2


def _tc_k2(s_p, a1p, dinv, Wc1, bc0):
    return pl.pallas_call(
        _k2_body,
        grid=(_G,),
        in_specs=[
            pl.BlockSpec((_NC, _BLK, _H), lambda i: (0, i, 0)),
            pl.BlockSpec((_BLK, _H), lambda i: (i, 0)),
            pl.BlockSpec((_BLK, 1), lambda i: (i, 0)),
            pl.BlockSpec((_H, _H), lambda i: (0, 0)),
            pl.BlockSpec((1, _H), lambda i: (0, 0)),
        ],
        out_specs=pl.BlockSpec((_BLK, _H), lambda i: (i, 0)),
        out_shape=jax.ShapeDtypeStruct((_N, _H), jnp.float32),
    )(s_p, a1p, dinv, Wc1, bc0)


def _k3_body(sp_ref, a2p_ref, dinv_ref, bc_ref, w2_ref, b2_ref, out_ref):
    dinv = dinv_ref[...]
    h2 = jnp.maximum(
        dinv * (sp_ref[0] + sp_ref[1] + a2p_ref[...]) + bc_ref[...], 0.0)
    logits = jnp.dot(h2, w2_ref[...],
                     preferred_element_type=jnp.float32) + b2_ref[...]
    m = jnp.max(logits, axis=-1, keepdims=True)
    e = jnp.exp(logits - m)
    lse = m + jnp.log(jnp.sum(e, axis=-1, keepdims=True))
    out_ref[...] = logits - lse


def _tc_k3(s_p, a2p, dinv, bc1, W2, b2):
    return pl.pallas_call(
        _k3_body,
        grid=(_G,),
        in_specs=[
            pl.BlockSpec((_NC, _BLK, _H), lambda i: (0, i, 0)),
            pl.BlockSpec((_BLK, _H), lambda i: (i, 0)),
            pl.BlockSpec((_BLK, 1), lambda i: (i, 0)),
            pl.BlockSpec((1, _H), lambda i: (0, 0)),
            pl.BlockSpec((_H, _C), lambda i: (0, 0)),
            pl.BlockSpec((1, _C), lambda i: (0, 0)),
        ],
        out_specs=pl.BlockSpec((_BLK, _C), lambda i: (i, 0)),
        out_shape=jax.ShapeDtypeStruct((_N, _C), jnp.float32),
    )(s_p, a2p, dinv, bc1, W2, b2)


def kernel(x, edge_index, edge_weight, W1, b1, Wc0, bc0, Wc1, bc1, W2, b2):
    src = edge_index[0]
    dst = edge_index[1]
    deg_p = _sc_deg(dst).reshape(_NC, _N, 1)
    a1p, dinv = _tc_k1(x, W1, b1.reshape(1, _H), Wc0, deg_p)
    s1p = _sc_agg(a1p, src, dst)
    a2p = _tc_k2(s1p, a1p, dinv, Wc1, bc0.reshape(1, _H))
    s2p = _sc_agg(a2p, src, dst)
    return _tc_k3(s2p, a2p, dinv, bc1.reshape(1, _H), W2, b2.reshape(1, _C))
